# Initial kernel scaffold; baseline (speedup 1.0000x reference)
import functools

import jax
import jax.numpy as jnp
from jax import lax
from jax.experimental import pallas as pl
from jax.experimental.pallas import tpu as pltpu
from jax.experimental.pallas import tpu_sc as plsc

N = 10000
E = 320000
D = 128
H = 128
G = 64

NC = 2
NS = 16
NW = NC * NS
EPT = E // NW
CHUNK = 80
NBUF = 5
NCHUNK = EPT // CHUNK
NOUTER = NCHUNK // NBUF

ROWS_A = 640
ROWS_LAST = N - (NS - 1) * ROWS_A

BLK = 1000
GRID = N // BLK


def _sc_mesh():
    return plsc.VectorSubcoreMesh(
        core_axis_name="c", subcore_axis_name="s", num_cores=NC, num_subcores=NS
    )


def _deg_body(dst_hbm, zeros_hbm, out_hbm, deg_sp, ones_v, idx_buf):
    cid = lax.axis_index("c")
    sid = lax.axis_index("s")
    wid = sid * NC + cid
    ebase = wid * EPT

    @pl.when(sid < NS - 1)
    def _():
        pltpu.sync_copy(zeros_hbm.at[pl.ds(sid * ROWS_A, ROWS_A)],
                        deg_sp.at[pl.ds(sid * ROWS_A, ROWS_A)])

    @pl.when(sid == NS - 1)
    def _():
        pltpu.sync_copy(zeros_hbm.at[pl.ds((NS - 1) * ROWS_A, ROWS_LAST)],
                        deg_sp.at[pl.ds((NS - 1) * ROWS_A, ROWS_LAST)])

    for j in range(CHUNK // 16):
        ones_v[pl.ds(j * 16, 16)] = jnp.ones((16,), jnp.float32)
    plsc.subcore_barrier()

    def chunk_step(c, carry):
        pltpu.sync_copy(dst_hbm.at[pl.ds(ebase + c * CHUNK, CHUNK)], idx_buf)
        pltpu.sync_copy(ones_v, deg_sp.at[idx_buf], add=True)
        return carry

    lax.fori_loop(0, NCHUNK, chunk_step, 0)
    plsc.subcore_barrier()

    @pl.when(sid < NS - 1)
    def _():
        pltpu.sync_copy(deg_sp.at[pl.ds(sid * ROWS_A, ROWS_A)],
                        out_hbm.at[cid, pl.ds(sid * ROWS_A, ROWS_A)])

    @pl.when(sid == NS - 1)
    def _():
        pltpu.sync_copy(deg_sp.at[pl.ds((NS - 1) * ROWS_A, ROWS_LAST)],
                        out_hbm.at[cid, pl.ds((NS - 1) * ROWS_A, ROWS_LAST)])


def _deg_partials(dst, zeros_n):
    f = pl.kernel(
        _deg_body,
        out_type=jax.ShapeDtypeStruct((NC, N), jnp.float32),
        mesh=_sc_mesh(),
        scratch_types=(
            pltpu.VMEM_SHARED((N,), jnp.float32),
            pltpu.VMEM((CHUNK,), jnp.float32),
            pltpu.VMEM((CHUNK,), jnp.int32),
        ),
        name="gcn_deg_sc",
    )
    return f(dst, zeros_n)


def _scatter_body(m_hbm, src_hbm, dst_hbm, zeros_hbm, out_hbm, agg, *bufs):
    src_bufs = bufs[:NBUF]
    dst_bufs = bufs[NBUF:2 * NBUF]
    row_bufs = bufs[2 * NBUF:3 * NBUF]
    sems = bufs[3 * NBUF:]
    cid = lax.axis_index("c")
    sid = lax.axis_index("s")
    wid = sid * NC + cid
    ebase = wid * EPT

    @pl.when(sid < NS - 1)
    def _():
        pltpu.sync_copy(zeros_hbm.at[pl.ds(sid * ROWS_A, ROWS_A)],
                        agg.at[pl.ds(sid * ROWS_A, ROWS_A)])

    @pl.when(sid == NS - 1)
    def _():
        pltpu.sync_copy(zeros_hbm.at[pl.ds((NS - 1) * ROWS_A, ROWS_LAST)],
                        agg.at[pl.ds((NS - 1) * ROWS_A, ROWS_LAST)])

    plsc.subcore_barrier()

    for b in range(NBUF):
        base = ebase + b * CHUNK
        pltpu.sync_copy(src_hbm.at[pl.ds(base, CHUNK)], src_bufs[b])
        pltpu.sync_copy(dst_hbm.at[pl.ds(base, CHUNK)], dst_bufs[b])
        pltpu.async_copy(m_hbm.at[src_bufs[b]], row_bufs[b], sems[b])

    def outer(o, carry):
        for b in range(NBUF):
            pltpu.make_async_copy(m_hbm.at[src_bufs[b]], row_bufs[b],
                                  sems[b]).wait()
            pltpu.sync_copy(row_bufs[b], agg.at[dst_bufs[b]], add=True)
            base = ebase + (o * NBUF + b + NBUF) * CHUNK
            pltpu.sync_copy(src_hbm.at[pl.ds(base, CHUNK)], src_bufs[b])
            pltpu.sync_copy(dst_hbm.at[pl.ds(base, CHUNK)], dst_bufs[b])
            pltpu.async_copy(m_hbm.at[src_bufs[b]], row_bufs[b], sems[b])
        return carry

    lax.fori_loop(0, NOUTER - 1, outer, 0)

    for b in range(NBUF):
        pltpu.make_async_copy(m_hbm.at[src_bufs[b]], row_bufs[b], sems[b]).wait()
        pltpu.sync_copy(row_bufs[b], agg.at[dst_bufs[b]], add=True)

    plsc.subcore_barrier()

    @pl.when(sid < NS - 1)
    def _():
        pltpu.sync_copy(agg.at[pl.ds(sid * ROWS_A, ROWS_A)],
                        out_hbm.at[cid, pl.ds(sid * ROWS_A, ROWS_A)])

    @pl.when(sid == NS - 1)
    def _():
        pltpu.sync_copy(agg.at[pl.ds((NS - 1) * ROWS_A, ROWS_LAST)],
                        out_hbm.at[cid, pl.ds((NS - 1) * ROWS_A, ROWS_LAST)])


def _scatter_partials(m, src, dst, zeros_m):
    f = pl.kernel(
        _scatter_body,
        out_type=jax.ShapeDtypeStruct((NC, N, H), jnp.float32),
        mesh=_sc_mesh(),
        scratch_types=(
            [pltpu.VMEM_SHARED((N, H), jnp.float32)]
            + [pltpu.VMEM((CHUNK,), jnp.int32) for _ in range(NBUF)]
            + [pltpu.VMEM((CHUNK,), jnp.int32) for _ in range(NBUF)]
            + [pltpu.VMEM((CHUNK, H), jnp.float32) for _ in range(NBUF)]
            + [pltpu.SemaphoreType.DMA for _ in range(NBUF)]
        ),
        name="gcn_scatter_sc",
    )
    return f(m, src, dst, zeros_m)


def _tc0_body(x_ref, w_ref, dinv_ref, out_ref):
    out_ref[...] = lax.dot_general(
        x_ref[...], w_ref[...], (((1,), (1,)), ((), ())),
        preferred_element_type=jnp.float32) * dinv_ref[...]


def _tc0(x, w, dinv2d):
    return pl.pallas_call(
        _tc0_body,
        grid=(GRID,),
        in_specs=[
            pl.BlockSpec((BLK, D), lambda i: (i, 0)),
            pl.BlockSpec((H, D), lambda i: (0, 0)),
            pl.BlockSpec((BLK, 1), lambda i: (i, 0)),
        ],
        out_specs=pl.BlockSpec((BLK, H), lambda i: (i, 0)),
        out_shape=jax.ShapeDtypeStruct((N, H), jnp.float32),
    )(x, w, dinv2d)


def _combine_h(m, pv, dinv, prm):
    agg = (pv[0] + pv[1] + m) * dinv
    b = prm[0:1, :]
    g = prm[1:2, :]
    bt = prm[2:3, :]
    rm = prm[3:4, :]
    rv = prm[4:5, :]
    h = (agg + b - rm) * lax.rsqrt(rv + 1e-5) * g + bt
    return jnp.maximum(h, 0.0)


def _tc_comb_body(m_ref, p_ref, dinv_ref, prm_ref, w_ref, out_ref):
    h = _combine_h(m_ref[...], p_ref[...], dinv_ref[...], prm_ref[...])
    out_ref[...] = lax.dot_general(
        h, w_ref[...], (((1,), (1,)), ((), ())),
        preferred_element_type=jnp.float32) * dinv_ref[...]


def _tc_combine(m, p, dinv2d, prm, w):
    return pl.pallas_call(
        _tc_comb_body,
        grid=(GRID,),
        in_specs=[
            pl.BlockSpec((BLK, H), lambda i: (i, 0)),
            pl.BlockSpec((NC, BLK, H), lambda i: (0, i, 0)),
            pl.BlockSpec((BLK, 1), lambda i: (i, 0)),
            pl.BlockSpec((5, H), lambda i: (0, 0)),
            pl.BlockSpec((H, H), lambda i: (0, 0)),
        ],
        out_specs=pl.BlockSpec((BLK, H), lambda i: (i, 0)),
        out_shape=jax.ShapeDtypeStruct((N, H), jnp.float32),
    )(m, p, dinv2d, prm, w)


def _tc_final_body(m_ref, p_ref, dinv_ref, prm_ref, batch_ref, wl_ref, bl_ref,
                   out_ref, acc_ref, cnt_ref):
    i = pl.program_id(0)

    @pl.when(i == 0)
    def _():
        acc_ref[...] = jnp.zeros_like(acc_ref)
        cnt_ref[...] = jnp.zeros_like(cnt_ref)

    h = _combine_h(m_ref[...], p_ref[...], dinv_ref[...], prm_ref[...])
    oh = (batch_ref[...] ==
          lax.broadcasted_iota(jnp.int32, (1, G), 1)).astype(jnp.float32)
    acc_ref[...] += lax.dot_general(
        oh, h, (((0,), (0,)), ((), ())), preferred_element_type=jnp.float32)
    cnt_ref[...] += lax.dot_general(
        oh, jnp.ones((BLK, 1), jnp.float32), (((0,), (0,)), ((), ())),
        preferred_element_type=jnp.float32)

    @pl.when(i == pl.num_programs(0) - 1)
    def _():
        pooled = acc_ref[...] / jnp.maximum(cnt_ref[...], 1.0)
        out_ref[...] = lax.dot_general(
            pooled, wl_ref[...], (((1,), (1,)), ((), ())),
            preferred_element_type=jnp.float32) + bl_ref[...]


def _tc_final(m, p, dinv2d, prm, batch2d, wl, bl2d):
    return pl.pallas_call(
        _tc_final_body,
        grid=(GRID,),
        in_specs=[
            pl.BlockSpec((BLK, H), lambda i: (i, 0)),
            pl.BlockSpec((NC, BLK, H), lambda i: (0, i, 0)),
            pl.BlockSpec((BLK, 1), lambda i: (i, 0)),
            pl.BlockSpec((5, H), lambda i: (0, 0)),
            pl.BlockSpec((BLK, 1), lambda i: (i, 0)),
            pl.BlockSpec((1, H), lambda i: (0, 0)),
            pl.BlockSpec((1, 1), lambda i: (0, 0)),
        ],
        out_specs=pl.BlockSpec((G, 1), lambda i: (0, 0)),
        out_shape=jax.ShapeDtypeStruct((G, 1), jnp.float32),
        scratch_shapes=[
            pltpu.VMEM((G, H), jnp.float32),
            pltpu.VMEM((G, 1), jnp.float32),
        ],
    )(m, p, dinv2d, prm, batch2d, wl, bl2d)


def kernel(x, edge_index, batch,
           W0, b0, g0, bt0, rm0, rv0,
           W1, b1, g1, bt1, rm1, rv1,
           W2, b2, g2, bt2, rm2, rv2,
           Wl, bl):
    src = edge_index[0]
    dst = edge_index[1]
    zeros_n = jnp.zeros((N,), jnp.float32)
    zeros_m = jnp.zeros((N, H), jnp.float32)

    degp = _deg_partials(dst, zeros_n)
    deg = degp[0] + degp[1] + 1.0
    dinv2d = lax.rsqrt(deg)[:, None]

    prm0 = jnp.stack([b0, g0, bt0, rm0, rv0])
    prm1 = jnp.stack([b1, g1, bt1, rm1, rv1])
    prm2 = jnp.stack([b2, g2, bt2, rm2, rv2])

    m0 = _tc0(x, W0, dinv2d)
    p0 = _scatter_partials(m0, src, dst, zeros_m)
    m1 = _tc_combine(m0, p0, dinv2d, prm0, W1)
    p1 = _scatter_partials(m1, src, dst, zeros_m)
    m2 = _tc_combine(m1, p1, dinv2d, prm1, W2)
    p2 = _scatter_partials(m2, src, dst, zeros_m)

    batch2d = batch[:, None]
    bl2d = bl[None, :]
    return _tc_final(m2, p2, dinv2d, prm2, batch2d, Wl, bl2d)


# speedup vs baseline: 17.5590x; 17.5590x over previous
import functools

import jax
import jax.numpy as jnp
from jax import lax
from jax.experimental import pallas as pl
from jax.experimental.pallas import tpu as pltpu
from jax.experimental.pallas import tpu_sc as plsc

N = 10000
E = 320000
D = 128
H = 128
G = 64

NC = 2
NS = 16
NW = NC * NS
EPT = E // NW
CHUNK = 80
NBUF = 4
NCHUNK = EPT // CHUNK
MAIN_OUTER = 30

RCHUNK = 80
NRCHUNK = N // RCHUNK
RR_ITERS = (NRCHUNK + NS - 1) // NS

BLK = 1000
GRID = N // BLK


def _sc_mesh():
    return plsc.VectorSubcoreMesh(
        core_axis_name="c", subcore_axis_name="s", num_cores=NC, num_subcores=NS
    )


def _deg_body(dst_hbm, out_hbm, deg_sp, ones_v, zbuf, idx_buf):
    cid = lax.axis_index("c")
    sid = lax.axis_index("s")
    wid = sid * NC + cid
    ebase = wid * EPT

    def zstore(i, carry):
        zbuf[pl.ds(i * 16, 16)] = jnp.zeros((16,), jnp.float32)
        return carry

    lax.fori_loop(0, RCHUNK // 16, zstore, 0)
    for j in range(CHUNK // 16):
        ones_v[pl.ds(j * 16, 16)] = jnp.ones((16,), jnp.float32)

    def zcopy(c, carry):
        k = c * NS + sid

        @pl.when(k < NRCHUNK)
        def _():
            pltpu.sync_copy(zbuf, deg_sp.at[pl.ds(k * RCHUNK, RCHUNK)])

        return carry

    lax.fori_loop(0, RR_ITERS, zcopy, 0)
    plsc.subcore_barrier()

    def chunk_step(c, carry):
        pltpu.sync_copy(dst_hbm.at[pl.ds(ebase + c * CHUNK, CHUNK)], idx_buf)
        pltpu.sync_copy(ones_v, deg_sp.at[idx_buf], add=True)
        return carry

    lax.fori_loop(0, NCHUNK, chunk_step, 0)
    plsc.subcore_barrier()

    def dump(c, carry):
        k = c * NS + sid

        @pl.when(k < NRCHUNK)
        def _():
            pltpu.sync_copy(deg_sp.at[pl.ds(k * RCHUNK, RCHUNK)], zbuf)
            pltpu.sync_copy(zbuf, out_hbm.at[pl.ds(cid * N + k * RCHUNK, RCHUNK)])

        return carry

    lax.fori_loop(0, RR_ITERS, dump, 0)


def _deg_partials(dst):
    f = pl.kernel(
        _deg_body,
        out_type=jax.ShapeDtypeStruct((NC * N,), jnp.float32),
        mesh=_sc_mesh(),
        scratch_types=(
            pltpu.VMEM_SHARED((N,), jnp.float32),
            pltpu.VMEM((CHUNK,), jnp.float32),
            pltpu.VMEM((RCHUNK,), jnp.float32),
            pltpu.VMEM((CHUNK,), jnp.int32),
        ),
        name="gcn_deg_sc",
    )
    return f(dst)


def _scatter_body(m_hbm, src_hbm, dst_hbm, out_hbm, agg, *bufs):
    src_bufs = bufs[:NBUF]
    dst_bufs = bufs[NBUF:2 * NBUF]
    row_bufs = bufs[2 * NBUF:3 * NBUF]
    sems = bufs[3 * NBUF:]
    cid = lax.axis_index("c")
    sid = lax.axis_index("s")
    wid = sid * NC + cid
    ebase = wid * EPT

    def zstore(r, carry):
        for j in range(H // 16):
            row_bufs[0][r, pl.ds(j * 16, 16)] = jnp.zeros((16,), jnp.float32)
        return carry

    lax.fori_loop(0, RCHUNK, zstore, 0)

    def zcopy(c, carry):
        k = c * NS + sid

        @pl.when(k < NRCHUNK)
        def _():
            pltpu.sync_copy(row_bufs[0], agg.at[pl.ds(k * RCHUNK, RCHUNK)])

        return carry

    lax.fori_loop(0, RR_ITERS, zcopy, 0)
    plsc.subcore_barrier()

    for b in range(NBUF):
        base = ebase + b * CHUNK
        pltpu.sync_copy(src_hbm.at[pl.ds(base, CHUNK)], src_bufs[b])
        pltpu.sync_copy(dst_hbm.at[pl.ds(base, CHUNK)], dst_bufs[b])
        pltpu.async_copy(m_hbm.at[src_bufs[b]], row_bufs[b], sems[b])

    def outer(o, carry):
        for b in range(NBUF):
            pltpu.make_async_copy(m_hbm.at[src_bufs[b]], row_bufs[b],
                                  sems[b]).wait()
            pltpu.sync_copy(row_bufs[b], agg.at[dst_bufs[b]], add=True)
            base = ebase + (o * NBUF + b + NBUF) * CHUNK
            pltpu.sync_copy(src_hbm.at[pl.ds(base, CHUNK)], src_bufs[b])
            pltpu.sync_copy(dst_hbm.at[pl.ds(base, CHUNK)], dst_bufs[b])
            pltpu.async_copy(m_hbm.at[src_bufs[b]], row_bufs[b], sems[b])
        return carry

    lax.fori_loop(0, MAIN_OUTER, outer, 0)

    for b in range(NBUF):
        pltpu.make_async_copy(m_hbm.at[src_bufs[b]], row_bufs[b], sems[b]).wait()
        pltpu.sync_copy(row_bufs[b], agg.at[dst_bufs[b]], add=True)

    base = ebase + (NCHUNK - 1) * CHUNK
    pltpu.sync_copy(src_hbm.at[pl.ds(base, CHUNK)], src_bufs[0])
    pltpu.sync_copy(dst_hbm.at[pl.ds(base, CHUNK)], dst_bufs[0])
    pltpu.async_copy(m_hbm.at[src_bufs[0]], row_bufs[0], sems[0]).wait()
    pltpu.sync_copy(row_bufs[0], agg.at[dst_bufs[0]], add=True)

    plsc.subcore_barrier()

    def dump(c, carry):
        k = c * NS + sid

        @pl.when(k < NRCHUNK)
        def _():
            b = 0
            pltpu.sync_copy(agg.at[pl.ds(k * RCHUNK, RCHUNK)], row_bufs[b])
            pltpu.sync_copy(row_bufs[b], out_hbm.at[cid, pl.ds(k * RCHUNK, RCHUNK)])

        return carry

    lax.fori_loop(0, RR_ITERS, dump, 0)


def _scatter_partials(m, src, dst):
    f = pl.kernel(
        _scatter_body,
        out_type=jax.ShapeDtypeStruct((NC, N, H), jnp.float32),
        mesh=_sc_mesh(),
        scratch_types=(
            [pltpu.VMEM_SHARED((N, H), jnp.float32)]
            + [pltpu.VMEM((CHUNK,), jnp.int32) for _ in range(NBUF)]
            + [pltpu.VMEM((CHUNK,), jnp.int32) for _ in range(NBUF)]
            + [pltpu.VMEM((CHUNK, H), jnp.float32) for _ in range(NBUF)]
            + [pltpu.SemaphoreType.DMA for _ in range(NBUF)]
        ),
        name="gcn_scatter_sc",
    )
    return f(m, src, dst)


def _tc0_body(x_ref, w_ref, dinv_ref, out_ref):
    out_ref[...] = lax.dot_general(
        x_ref[...], w_ref[...], (((1,), (1,)), ((), ())),
        preferred_element_type=jnp.float32) * dinv_ref[...]


def _tc0(x, w, dinv2d):
    return pl.pallas_call(
        _tc0_body,
        grid=(GRID,),
        in_specs=[
            pl.BlockSpec((BLK, D), lambda i: (i, 0)),
            pl.BlockSpec((H, D), lambda i: (0, 0)),
            pl.BlockSpec((BLK, 1), lambda i: (i, 0)),
        ],
        out_specs=pl.BlockSpec((BLK, H), lambda i: (i, 0)),
        out_shape=jax.ShapeDtypeStruct((N, H), jnp.float32),
    )(x, w, dinv2d)


def _combine_h(m, pv, dinv, prm):
    agg = (pv[0] + pv[1] + m) * dinv
    b = prm[0:1, :]
    g = prm[1:2, :]
    bt = prm[2:3, :]
    rm = prm[3:4, :]
    rv = prm[4:5, :]
    h = (agg + b - rm) * lax.rsqrt(rv + 1e-5) * g + bt
    return jnp.maximum(h, 0.0)


def _tc_comb_body(m_ref, p_ref, dinv_ref, prm_ref, w_ref, out_ref):
    h = _combine_h(m_ref[...], p_ref[...], dinv_ref[...], prm_ref[...])
    out_ref[...] = lax.dot_general(
        h, w_ref[...], (((1,), (1,)), ((), ())),
        preferred_element_type=jnp.float32) * dinv_ref[...]


def _tc_combine(m, p, dinv2d, prm, w):
    return pl.pallas_call(
        _tc_comb_body,
        grid=(GRID,),
        in_specs=[
            pl.BlockSpec((BLK, H), lambda i: (i, 0)),
            pl.BlockSpec((NC, BLK, H), lambda i: (0, i, 0)),
            pl.BlockSpec((BLK, 1), lambda i: (i, 0)),
            pl.BlockSpec((5, H), lambda i: (0, 0)),
            pl.BlockSpec((H, H), lambda i: (0, 0)),
        ],
        out_specs=pl.BlockSpec((BLK, H), lambda i: (i, 0)),
        out_shape=jax.ShapeDtypeStruct((N, H), jnp.float32),
    )(m, p, dinv2d, prm, w)


def _tc_final_body(m_ref, p_ref, dinv_ref, prm_ref, batch_ref, wl_ref,
                   out_ref, acc_ref, cnt_ref):
    i = pl.program_id(0)

    @pl.when(i == 0)
    def _():
        acc_ref[...] = jnp.zeros_like(acc_ref)
        cnt_ref[...] = jnp.zeros_like(cnt_ref)

    h = _combine_h(m_ref[...], p_ref[...], dinv_ref[...], prm_ref[...])
    oh = (batch_ref[...] ==
          lax.broadcasted_iota(jnp.int32, (1, G), 1)).astype(jnp.float32)
    acc_ref[...] += lax.dot_general(
        oh, h, (((0,), (0,)), ((), ())), preferred_element_type=jnp.float32)
    cnt_ref[...] += lax.dot_general(
        oh, jnp.ones((BLK, H), jnp.float32), (((0,), (0,)), ((), ())),
        preferred_element_type=jnp.float32)

    @pl.when(i == pl.num_programs(0) - 1)
    def _():
        pooled = acc_ref[...] / jnp.maximum(cnt_ref[...], 1.0)
        out_ref[...] = lax.dot_general(
            wl_ref[...], pooled, (((1,), (1,)), ((), ())),
            preferred_element_type=jnp.float32)


def _tc_final(m, p, dinv2d, prm, batch2d, wl):
    return pl.pallas_call(
        _tc_final_body,
        grid=(GRID,),
        in_specs=[
            pl.BlockSpec((BLK, H), lambda i: (i, 0)),
            pl.BlockSpec((NC, BLK, H), lambda i: (0, i, 0)),
            pl.BlockSpec((BLK, 1), lambda i: (i, 0)),
            pl.BlockSpec((5, H), lambda i: (0, 0)),
            pl.BlockSpec((BLK, 1), lambda i: (i, 0)),
            pl.BlockSpec((1, H), lambda i: (0, 0)),
        ],
        out_specs=pl.BlockSpec((1, G), lambda i: (0, 0)),
        out_shape=jax.ShapeDtypeStruct((1, G), jnp.float32),
        scratch_shapes=[
            pltpu.VMEM((G, H), jnp.float32),
            pltpu.VMEM((G, H), jnp.float32),
        ],
    )(m, p, dinv2d, prm, batch2d, wl)


def kernel(x, edge_index, batch,
           W0, b0, g0, bt0, rm0, rv0,
           W1, b1, g1, bt1, rm1, rv1,
           W2, b2, g2, bt2, rm2, rv2,
           Wl, bl):
    src = edge_index[0]
    dst = edge_index[1]

    degp = _deg_partials(dst).reshape(NC, N)
    deg = degp[0] + degp[1] + 1.0
    dinv2d = lax.rsqrt(deg)[:, None]

    prm0 = jnp.stack([b0, g0, bt0, rm0, rv0])
    prm1 = jnp.stack([b1, g1, bt1, rm1, rv1])
    prm2 = jnp.stack([b2, g2, bt2, rm2, rv2])

    m0 = _tc0(x, W0, dinv2d)
    p0 = _scatter_partials(m0, src, dst)
    m1 = _tc_combine(m0, p0, dinv2d, prm0, W1)
    p1 = _scatter_partials(m1, src, dst)
    m2 = _tc_combine(m1, p1, dinv2d, prm1, W2)
    p2 = _scatter_partials(m2, src, dst)

    batch2d = batch[:, None]
    out = _tc_final(m2, p2, dinv2d, prm2, batch2d, Wl)
    return out.reshape(G, 1) + bl


# speedup vs baseline: 20.7307x; 1.1806x over previous
import functools

import jax
import jax.numpy as jnp
from jax import lax
from jax.experimental import pallas as pl
from jax.experimental.pallas import tpu as pltpu
from jax.experimental.pallas import tpu_sc as plsc

N = 10000
E = 320000
D = 128
H = 128
G = 64

NC = 2
NS = 16
NW = NC * NS
EPT = E // NW
CHUNK = 80
M = EPT // CHUNK
RBUF = 4
LBUF = 8

NRCH = N // CHUNK
RR_IT = (NRCH + NS - 1) // NS

BLK = 1000
GRID = N // BLK


def _sc_mesh():
    return plsc.VectorSubcoreMesh(
        core_axis_name="c", subcore_axis_name="s", num_cores=NC, num_subcores=NS
    )


def _deg_body(packed_hbm, dst_hbm, out_hbm, deg_sp, ones_v, zbuf, *scr):
    lbufs = scr[:RBUF]
    lsem = scr[RBUF:2 * RBUF]
    ssem = scr[2 * RBUF:3 * RBUF]
    cid = lax.axis_index("c")
    sid = lax.axis_index("s")
    wid = sid * NC + cid
    rowbase = wid * M

    def zstore(i, carry):
        zbuf[pl.ds(i * 16, 16)] = jnp.zeros((16,), jnp.float32)
        return carry

    lax.fori_loop(0, 640 // 16, zstore, 0)
    for j in range(CHUNK // 16):
        ones_v[pl.ds(j * 16, 16)] = jnp.ones((16,), jnp.float32)

    @pl.when(sid < NS - 1)
    def _():
        pltpu.sync_copy(zbuf, deg_sp.at[pl.ds(sid * 640, 640)])

    @pl.when(sid == NS - 1)
    def _():
        pltpu.sync_copy(zbuf.at[pl.ds(0, 400)],
                        deg_sp.at[pl.ds((NS - 1) * 640, 400)])

    plsc.subcore_barrier()

    def issue_l(c, ls):
        pltpu.async_copy(packed_hbm.at[rowbase + c], lbufs[ls], lsem[ls])

    def wait_l(ls):
        pltpu.make_async_copy(packed_hbm.at[0], lbufs[ls], lsem[ls]).wait()

    def issue_s(ls):
        pltpu.async_copy(ones_v, deg_sp.at[lbufs[ls].at[1]], ssem[ls],
                         add=True)

    def wait_s(ls):
        pltpu.make_async_copy(ones_v, deg_sp.at[lbufs[ls].at[1]],
                              ssem[ls]).wait()

    issue_l(0, 0)
    issue_l(1, 1)
    for c in (0, 1):
        wait_l(c % RBUF)
        issue_s(c % RBUF)
        issue_l(c + 2, (c + 2) % RBUF)

    def outer(o, carry):
        for r in range(RBUF):
            c = 2 + o * RBUF + r
            k = (2 + r) % RBUF
            wait_l(k)
            issue_s(k)
            wait_s((k + 2) % RBUF)
            issue_l(c + 2, (k + 2) % RBUF)
        return carry

    lax.fori_loop(0, (122 - 2) // RBUF, outer, 0)

    for c in range(122, M):
        wait_l(c % RBUF)
        issue_s(c % RBUF)
        wait_s((c + 2) % RBUF)
        if c + 2 < M:
            issue_l(c + 2, (c + 2) % RBUF)
    wait_s((M - 2) % RBUF)
    wait_s((M - 1) % RBUF)

    plsc.subcore_barrier()

    @pl.when(sid < NS - 1)
    def _():
        pltpu.sync_copy(deg_sp.at[pl.ds(sid * 640, 640)], zbuf)
        pltpu.sync_copy(zbuf, out_hbm.at[pl.ds(cid * N + sid * 640, 640)])

    @pl.when(sid == NS - 1)
    def _():
        pltpu.sync_copy(deg_sp.at[pl.ds((NS - 1) * 640, 400)],
                        zbuf.at[pl.ds(0, 400)])
        pltpu.sync_copy(zbuf.at[pl.ds(0, 400)],
                        out_hbm.at[pl.ds(cid * N + (NS - 1) * 640, 400)])


def _deg_partials(packed, dst):
    f = pl.kernel(
        _deg_body,
        out_type=jax.ShapeDtypeStruct((NC * N,), jnp.float32),
        mesh=_sc_mesh(),
        scratch_types=(
            [pltpu.VMEM_SHARED((N,), jnp.float32),
             pltpu.VMEM((CHUNK,), jnp.float32),
             pltpu.VMEM((640,), jnp.float32)]
            + [pltpu.VMEM((2, CHUNK), jnp.int32) for _ in range(RBUF)]
            + [pltpu.SemaphoreType.DMA for _ in range(2 * RBUF)]
        ),
        name="gcn_deg_sc",
    )
    return f(packed, dst)


def _scatter_body(m_hbm, packed_hbm, src_hbm, dst_hbm, out_hbm, agg, *scr):
    lbufs = scr[:LBUF]
    rbufs = scr[LBUF:LBUF + RBUF]
    base = LBUF + RBUF
    lsem = scr[base:base + LBUF]
    gsem = scr[base + LBUF:base + LBUF + RBUF]
    ssem = scr[base + LBUF + RBUF:base + LBUF + 2 * RBUF]
    cid = lax.axis_index("c")
    sid = lax.axis_index("s")
    wid = sid * NC + cid
    rowbase = wid * M

    def zstore(r, carry):
        for j in range(H // 16):
            rbufs[0][r, pl.ds(j * 16, 16)] = jnp.zeros((16,), jnp.float32)
        return carry

    lax.fori_loop(0, CHUNK, zstore, 0)

    def zcopy(c, carry):
        k = c * NS + sid

        @pl.when(k < NRCH)
        def _():
            pltpu.sync_copy(rbufs[0], agg.at[pl.ds(k * CHUNK, CHUNK)])

        return carry

    lax.fori_loop(0, RR_IT, zcopy, 0)
    plsc.subcore_barrier()

    def issue_l(c, ls):
        pltpu.async_copy(packed_hbm.at[rowbase + c], lbufs[ls], lsem[ls])

    def wait_l(ls):
        pltpu.make_async_copy(packed_hbm.at[0], lbufs[ls], lsem[ls]).wait()

    def issue_g(ls, rs):
        pltpu.async_copy(m_hbm.at[lbufs[ls].at[0]], rbufs[rs], gsem[rs])

    def wait_g(ls, rs):
        pltpu.make_async_copy(m_hbm.at[lbufs[ls].at[0]], rbufs[rs],
                              gsem[rs]).wait()

    def issue_s(ls, rs):
        pltpu.async_copy(rbufs[rs], agg.at[lbufs[ls].at[1]], ssem[rs],
                         add=True)

    def wait_s(ls, rs):
        pltpu.make_async_copy(rbufs[rs], agg.at[lbufs[ls].at[1]],
                              ssem[rs]).wait()

    def body(c, k, do_a=True, do_b=True, do_c=True):
        if do_a:
            wait_g((k - 1) % LBUF, (k - 1) % RBUF)
            issue_s((k - 1) % LBUF, (k - 1) % RBUF)
        if do_b:
            if do_a or k >= 4:
                wait_s((k - 4) % LBUF, k % RBUF)
            wait_l(k % LBUF)
            issue_g(k % LBUF, k % RBUF)
        if do_c:
            issue_l(c + 3, (k + 3) % LBUF)

    issue_l(0, 0)
    issue_l(1, 1)
    issue_l(2, 2)
    body(0, 0, do_a=False, do_b=True, do_c=True)
    for c in (1, 2, 3):
        wait_g(c - 1, c - 1)
        issue_s(c - 1, c - 1)
        wait_l(c)
        issue_g(c, c)
        issue_l(c + 3, c + 3)

    def outer(o, carry):
        for r in range(LBUF):
            c = 4 + o * LBUF + r
            body(c, (4 + r) % LBUF)
        return carry

    lax.fori_loop(0, (116 - 4) // LBUF, outer, 0)

    for c in range(116, M):
        k = c % LBUF
        wait_g((k - 1) % LBUF, (k - 1) % RBUF)
        issue_s((k - 1) % LBUF, (k - 1) % RBUF)
        wait_s((k - 4) % LBUF, k % RBUF)
        wait_l(k)
        issue_g(k, k % RBUF)
        if c + 3 < M:
            issue_l(c + 3, (k + 3) % LBUF)
    wait_g((M - 1) % LBUF, (M - 1) % RBUF)
    issue_s((M - 1) % LBUF, (M - 1) % RBUF)
    for c in range(M - RBUF, M):
        wait_s(c % LBUF, c % RBUF)

    plsc.subcore_barrier()

    def dump(c, carry):
        k = c * NS + sid

        @pl.when(k < NRCH)
        def _():
            pltpu.sync_copy(agg.at[pl.ds(k * CHUNK, CHUNK)], rbufs[0])
            pltpu.sync_copy(rbufs[0], out_hbm.at[cid, pl.ds(k * CHUNK, CHUNK)])

        return carry

    lax.fori_loop(0, RR_IT, dump, 0)


def _scatter_partials(m, packed, src, dst):
    f = pl.kernel(
        _scatter_body,
        out_type=jax.ShapeDtypeStruct((NC, N, H), jnp.float32),
        mesh=_sc_mesh(),
        scratch_types=(
            [pltpu.VMEM_SHARED((N, H), jnp.float32)]
            + [pltpu.VMEM((2, CHUNK), jnp.int32) for _ in range(LBUF)]
            + [pltpu.VMEM((CHUNK, H), jnp.float32) for _ in range(RBUF)]
            + [pltpu.SemaphoreType.DMA for _ in range(LBUF + 2 * RBUF)]
        ),
        name="gcn_scatter_sc",
    )
    return f(m, packed, src, dst)


def _tc0_body(x_ref, w_ref, dinv_ref, out_ref):
    out_ref[...] = lax.dot_general(
        x_ref[...], w_ref[...], (((1,), (1,)), ((), ())),
        preferred_element_type=jnp.float32) * dinv_ref[...]


def _tc0(x, w, dinv2d):
    return pl.pallas_call(
        _tc0_body,
        grid=(GRID,),
        in_specs=[
            pl.BlockSpec((BLK, D), lambda i: (i, 0)),
            pl.BlockSpec((H, D), lambda i: (0, 0)),
            pl.BlockSpec((BLK, 1), lambda i: (i, 0)),
        ],
        out_specs=pl.BlockSpec((BLK, H), lambda i: (i, 0)),
        out_shape=jax.ShapeDtypeStruct((N, H), jnp.float32),
    )(x, w, dinv2d)


def _combine_h(m, pv, dinv, prm):
    agg = (pv[0] + pv[1] + m) * dinv
    b = prm[0:1, :]
    g = prm[1:2, :]
    bt = prm[2:3, :]
    rm = prm[3:4, :]
    rv = prm[4:5, :]
    h = (agg + b - rm) * lax.rsqrt(rv + 1e-5) * g + bt
    return jnp.maximum(h, 0.0)


def _tc_comb_body(m_ref, p_ref, dinv_ref, prm_ref, w_ref, out_ref):
    h = _combine_h(m_ref[...], p_ref[...], dinv_ref[...], prm_ref[...])
    out_ref[...] = lax.dot_general(
        h, w_ref[...], (((1,), (1,)), ((), ())),
        preferred_element_type=jnp.float32) * dinv_ref[...]


def _tc_combine(m, p, dinv2d, prm, w):
    return pl.pallas_call(
        _tc_comb_body,
        grid=(GRID,),
        in_specs=[
            pl.BlockSpec((BLK, H), lambda i: (i, 0)),
            pl.BlockSpec((NC, BLK, H), lambda i: (0, i, 0)),
            pl.BlockSpec((BLK, 1), lambda i: (i, 0)),
            pl.BlockSpec((5, H), lambda i: (0, 0)),
            pl.BlockSpec((H, H), lambda i: (0, 0)),
        ],
        out_specs=pl.BlockSpec((BLK, H), lambda i: (i, 0)),
        out_shape=jax.ShapeDtypeStruct((N, H), jnp.float32),
    )(m, p, dinv2d, prm, w)


def _tc_final_body(m_ref, p_ref, dinv_ref, prm_ref, batch_ref, wl_ref,
                   out_ref, acc_ref, cnt_ref):
    i = pl.program_id(0)

    @pl.when(i == 0)
    def _():
        acc_ref[...] = jnp.zeros_like(acc_ref)
        cnt_ref[...] = jnp.zeros_like(cnt_ref)

    h = _combine_h(m_ref[...], p_ref[...], dinv_ref[...], prm_ref[...])
    oh = (batch_ref[...] ==
          lax.broadcasted_iota(jnp.int32, (1, G), 1)).astype(jnp.float32)
    acc_ref[...] += lax.dot_general(
        oh, h, (((0,), (0,)), ((), ())), preferred_element_type=jnp.float32)
    cnt_ref[...] += lax.dot_general(
        oh, jnp.ones((BLK, H), jnp.float32), (((0,), (0,)), ((), ())),
        preferred_element_type=jnp.float32)

    @pl.when(i == pl.num_programs(0) - 1)
    def _():
        pooled = acc_ref[...] / jnp.maximum(cnt_ref[...], 1.0)
        out_ref[...] = lax.dot_general(
            wl_ref[...], pooled, (((1,), (1,)), ((), ())),
            preferred_element_type=jnp.float32)


def _tc_final(m, p, dinv2d, prm, batch2d, wl):
    return pl.pallas_call(
        _tc_final_body,
        grid=(GRID,),
        in_specs=[
            pl.BlockSpec((BLK, H), lambda i: (i, 0)),
            pl.BlockSpec((NC, BLK, H), lambda i: (0, i, 0)),
            pl.BlockSpec((BLK, 1), lambda i: (i, 0)),
            pl.BlockSpec((5, H), lambda i: (0, 0)),
            pl.BlockSpec((BLK, 1), lambda i: (i, 0)),
            pl.BlockSpec((1, H), lambda i: (0, 0)),
        ],
        out_specs=pl.BlockSpec((1, G), lambda i: (0, 0)),
        out_shape=jax.ShapeDtypeStruct((1, G), jnp.float32),
        scratch_shapes=[
            pltpu.VMEM((G, H), jnp.float32),
            pltpu.VMEM((G, H), jnp.float32),
        ],
    )(m, p, dinv2d, prm, batch2d, wl)


def kernel(x, edge_index, batch,
           W0, b0, g0, bt0, rm0, rv0,
           W1, b1, g1, bt1, rm1, rv1,
           W2, b2, g2, bt2, rm2, rv2,
           Wl, bl):
    src = edge_index[0]
    dst = edge_index[1]
    srcm = src.reshape(NW, EPT)[:, :M * CHUNK].reshape(NW, M, CHUNK)
    dstm = dst.reshape(NW, EPT)[:, :M * CHUNK].reshape(NW, M, CHUNK)
    packed = jnp.stack([srcm, dstm], axis=2).reshape(NW * M, 2, CHUNK)

    degp = _deg_partials(packed, dst).reshape(NC, N)
    deg = degp[0] + degp[1] + 1.0
    dinv2d = lax.rsqrt(deg)[:, None]

    prm0 = jnp.stack([b0, g0, bt0, rm0, rv0])
    prm1 = jnp.stack([b1, g1, bt1, rm1, rv1])
    prm2 = jnp.stack([b2, g2, bt2, rm2, rv2])

    m0 = _tc0(x, W0, dinv2d)
    p0 = _scatter_partials(m0, packed, src, dst)
    m1 = _tc_combine(m0, p0, dinv2d, prm0, W1)
    p1 = _scatter_partials(m1, packed, src, dst)
    m2 = _tc_combine(m1, p1, dinv2d, prm1, W2)
    p2 = _scatter_partials(m2, packed, src, dst)

    batch2d = batch[:, None]
    out = _tc_final(m2, p2, dinv2d, prm2, batch2d, Wl)
    return out.reshape(G, 1) + bl


# speedup vs baseline: 21.0035x; 1.0132x over previous
import functools

import jax
import jax.numpy as jnp
from jax import lax
from jax.experimental import pallas as pl
from jax.experimental.pallas import tpu as pltpu
from jax.experimental.pallas import tpu_sc as plsc

N = 10000
E = 320000
D = 128
H = 128
G = 64

NC = 2
NS = 16
NW = NC * NS
EPT = E // NW
CHUNK = 80
M = EPT // CHUNK
RBUF = 4
LBUF = 8

NRCH = N // CHUNK
RR_IT = (NRCH + NS - 1) // NS

BLK = 1000
GRID = N // BLK


def _sc_mesh():
    return plsc.VectorSubcoreMesh(
        core_axis_name="c", subcore_axis_name="s", num_cores=NC, num_subcores=NS
    )


def _deg_body(packed_hbm, dst_hbm, out_hbm, deg_sp, ones_v, zbuf, *scr):
    lbufs = scr[:RBUF]
    lsem = scr[RBUF:2 * RBUF]
    ssem = scr[2 * RBUF:3 * RBUF]
    cid = lax.axis_index("c")
    sid = lax.axis_index("s")
    wid = sid * NC + cid
    rowbase = wid * M

    def zstore(i, carry):
        zbuf[pl.ds(i * 16, 16)] = jnp.zeros((16,), jnp.float32)
        return carry

    lax.fori_loop(0, 640 // 16, zstore, 0)
    for j in range(CHUNK // 16):
        ones_v[pl.ds(j * 16, 16)] = jnp.ones((16,), jnp.float32)

    @pl.when(sid < NS - 1)
    def _():
        pltpu.sync_copy(zbuf, deg_sp.at[pl.ds(sid * 640, 640)])

    @pl.when(sid == NS - 1)
    def _():
        pltpu.sync_copy(zbuf.at[pl.ds(0, 400)],
                        deg_sp.at[pl.ds((NS - 1) * 640, 400)])

    plsc.subcore_barrier()

    def issue_l(c, ls):
        pltpu.async_copy(packed_hbm.at[rowbase + c], lbufs[ls], lsem[ls])

    def wait_l(ls):
        pltpu.make_async_copy(packed_hbm.at[0], lbufs[ls], lsem[ls]).wait()

    def issue_s(ls):
        pltpu.async_copy(ones_v, deg_sp.at[lbufs[ls].at[1]], ssem[ls],
                         add=True)

    def wait_s(ls):
        pltpu.make_async_copy(ones_v, deg_sp.at[lbufs[ls].at[1]],
                              ssem[ls]).wait()

    issue_l(0, 0)
    issue_l(1, 1)
    for c in (0, 1):
        wait_l(c % RBUF)
        issue_s(c % RBUF)
        issue_l(c + 2, (c + 2) % RBUF)

    def outer(o, carry):
        for r in range(RBUF):
            c = 2 + o * RBUF + r
            k = (2 + r) % RBUF
            wait_l(k)
            issue_s(k)
            wait_s((k + 2) % RBUF)
            issue_l(c + 2, (k + 2) % RBUF)
        return carry

    lax.fori_loop(0, (122 - 2) // RBUF, outer, 0)

    for c in range(122, M):
        wait_l(c % RBUF)
        issue_s(c % RBUF)
        wait_s((c + 2) % RBUF)
        if c + 2 < M:
            issue_l(c + 2, (c + 2) % RBUF)
    wait_s((M - 2) % RBUF)
    wait_s((M - 1) % RBUF)

    plsc.subcore_barrier()

    @pl.when(sid < NS - 1)
    def _():
        pltpu.sync_copy(deg_sp.at[pl.ds(sid * 640, 640)], zbuf)
        pltpu.sync_copy(zbuf, out_hbm.at[pl.ds(cid * N + sid * 640, 640)])

    @pl.when(sid == NS - 1)
    def _():
        pltpu.sync_copy(deg_sp.at[pl.ds((NS - 1) * 640, 400)],
                        zbuf.at[pl.ds(0, 400)])
        pltpu.sync_copy(zbuf.at[pl.ds(0, 400)],
                        out_hbm.at[pl.ds(cid * N + (NS - 1) * 640, 400)])


def _deg_partials(packed, dst):
    f = pl.kernel(
        _deg_body,
        out_type=jax.ShapeDtypeStruct((NC * N,), jnp.float32),
        mesh=_sc_mesh(),
        scratch_types=(
            [pltpu.VMEM_SHARED((N,), jnp.float32),
             pltpu.VMEM((CHUNK,), jnp.float32),
             pltpu.VMEM((640,), jnp.float32)]
            + [pltpu.VMEM((2, CHUNK), jnp.int32) for _ in range(RBUF)]
            + [pltpu.SemaphoreType.DMA for _ in range(2 * RBUF)]
        ),
        name="gcn_deg_sc",
    )
    return f(packed, dst)


def _scatter_body(m_hbm, packed_hbm, src_hbm, dst_hbm, out_hbm, agg, *scr):
    lbufs = scr[:LBUF]
    rbufs = scr[LBUF:LBUF + RBUF]
    base = LBUF + RBUF
    lsem = scr[base:base + LBUF]
    gsem = scr[base + LBUF:base + LBUF + RBUF]
    ssem = scr[base + LBUF + RBUF:base + LBUF + 2 * RBUF]
    cid = lax.axis_index("c")
    sid = lax.axis_index("s")
    wid = sid * NC + cid
    rowbase = wid * M

    def zstore(r, carry):
        for j in range(H // 16):
            rbufs[0][r, pl.ds(j * 16, 16)] = jnp.zeros((16,), jnp.float32)
        return carry

    lax.fori_loop(0, CHUNK, zstore, 0)

    for c in range(RR_IT):
        k = c * NS + sid

        @pl.when(k < NRCH)
        def _():
            pltpu.async_copy(rbufs[0], agg.at[pl.ds(k * CHUNK, CHUNK)],
                             gsem[0])

    for c in range(RR_IT):
        k = c * NS + sid

        @pl.when(k < NRCH)
        def _():
            pltpu.make_async_copy(rbufs[0], agg.at[pl.ds(k * CHUNK, CHUNK)],
                                  gsem[0]).wait()

    plsc.subcore_barrier()

    def issue_l(c, ls):
        pltpu.async_copy(packed_hbm.at[rowbase + c], lbufs[ls], lsem[ls])

    def wait_l(ls):
        pltpu.make_async_copy(packed_hbm.at[0], lbufs[ls], lsem[ls]).wait()

    def issue_g(ls, rs):
        pltpu.async_copy(m_hbm.at[lbufs[ls].at[0]], rbufs[rs], gsem[rs])

    def wait_g(ls, rs):
        pltpu.make_async_copy(m_hbm.at[lbufs[ls].at[0]], rbufs[rs],
                              gsem[rs]).wait()

    def issue_s(ls, rs):
        pltpu.async_copy(rbufs[rs], agg.at[lbufs[ls].at[1]], ssem[rs],
                         add=True)

    def wait_s(ls, rs):
        pltpu.make_async_copy(rbufs[rs], agg.at[lbufs[ls].at[1]],
                              ssem[rs]).wait()

    def body(c, k, do_a=True, do_b=True, do_c=True):
        if do_a:
            wait_g((k - 1) % LBUF, (k - 1) % RBUF)
            issue_s((k - 1) % LBUF, (k - 1) % RBUF)
        if do_b:
            if do_a or k >= 4:
                wait_s((k - 4) % LBUF, k % RBUF)
            wait_l(k % LBUF)
            issue_g(k % LBUF, k % RBUF)
        if do_c:
            issue_l(c + 3, (k + 3) % LBUF)

    issue_l(0, 0)
    issue_l(1, 1)
    issue_l(2, 2)
    body(0, 0, do_a=False, do_b=True, do_c=True)
    for c in (1, 2, 3):
        wait_g(c - 1, c - 1)
        issue_s(c - 1, c - 1)
        wait_l(c)
        issue_g(c, c)
        issue_l(c + 3, c + 3)

    def outer(o, carry):
        for r in range(LBUF):
            c = 4 + o * LBUF + r
            body(c, (4 + r) % LBUF)
        return carry

    lax.fori_loop(0, (116 - 4) // LBUF, outer, 0)

    for c in range(116, M):
        k = c % LBUF
        wait_g((k - 1) % LBUF, (k - 1) % RBUF)
        issue_s((k - 1) % LBUF, (k - 1) % RBUF)
        wait_s((k - 4) % LBUF, k % RBUF)
        wait_l(k)
        issue_g(k, k % RBUF)
        if c + 3 < M:
            issue_l(c + 3, (k + 3) % LBUF)
    wait_g((M - 1) % LBUF, (M - 1) % RBUF)
    issue_s((M - 1) % LBUF, (M - 1) % RBUF)
    for c in range(M - RBUF, M):
        wait_s(c % LBUF, c % RBUF)

    plsc.subcore_barrier()

    for c in range(RR_IT):
        k = c * NS + sid
        b = c % 2

        @pl.when(k < NRCH)
        def _():
            if c >= 2:
                kp = (c - 2) * NS + sid
                pltpu.make_async_copy(
                    rbufs[b], out_hbm.at[cid, pl.ds(kp * CHUNK, CHUNK)],
                    ssem[b]).wait()
            pltpu.sync_copy(agg.at[pl.ds(k * CHUNK, CHUNK)], rbufs[b])
            pltpu.async_copy(rbufs[b], out_hbm.at[cid, pl.ds(k * CHUNK, CHUNK)],
                             ssem[b])

    pltpu.make_async_copy(rbufs[0], out_hbm.at[cid, pl.ds(0, CHUNK)],
                          ssem[0]).wait()
    pltpu.make_async_copy(rbufs[1], out_hbm.at[cid, pl.ds(0, CHUNK)],
                          ssem[1]).wait()


def _scatter_partials(m, packed, src, dst):
    f = pl.kernel(
        _scatter_body,
        out_type=jax.ShapeDtypeStruct((NC, N, H), jnp.float32),
        mesh=_sc_mesh(),
        scratch_types=(
            [pltpu.VMEM_SHARED((N, H), jnp.float32)]
            + [pltpu.VMEM((2, CHUNK), jnp.int32) for _ in range(LBUF)]
            + [pltpu.VMEM((CHUNK, H), jnp.float32) for _ in range(RBUF)]
            + [pltpu.SemaphoreType.DMA for _ in range(LBUF + 2 * RBUF)]
        ),
        name="gcn_scatter_sc",
    )
    return f(m, packed, src, dst)


def _tc0_body(x_ref, w_ref, dinv_ref, out_ref):
    out_ref[...] = lax.dot_general(
        x_ref[...], w_ref[...], (((1,), (1,)), ((), ())),
        preferred_element_type=jnp.float32) * dinv_ref[...]


def _tc0(x, w, dinv2d):
    return pl.pallas_call(
        _tc0_body,
        grid=(GRID,),
        in_specs=[
            pl.BlockSpec((BLK, D), lambda i: (i, 0)),
            pl.BlockSpec((H, D), lambda i: (0, 0)),
            pl.BlockSpec((BLK, 1), lambda i: (i, 0)),
        ],
        out_specs=pl.BlockSpec((BLK, H), lambda i: (i, 0)),
        out_shape=jax.ShapeDtypeStruct((N, H), jnp.float32),
    )(x, w, dinv2d)


def _combine_h(m, pv, dinv, prm):
    agg = (pv[0] + pv[1] + m) * dinv
    b = prm[0:1, :]
    g = prm[1:2, :]
    bt = prm[2:3, :]
    rm = prm[3:4, :]
    rv = prm[4:5, :]
    h = (agg + b - rm) * lax.rsqrt(rv + 1e-5) * g + bt
    return jnp.maximum(h, 0.0)


def _tc_comb_body(m_ref, p_ref, dinv_ref, prm_ref, w_ref, out_ref):
    h = _combine_h(m_ref[...], p_ref[...], dinv_ref[...], prm_ref[...])
    out_ref[...] = lax.dot_general(
        h, w_ref[...], (((1,), (1,)), ((), ())),
        preferred_element_type=jnp.float32) * dinv_ref[...]


def _tc_combine(m, p, dinv2d, prm, w):
    return pl.pallas_call(
        _tc_comb_body,
        grid=(GRID,),
        in_specs=[
            pl.BlockSpec((BLK, H), lambda i: (i, 0)),
            pl.BlockSpec((NC, BLK, H), lambda i: (0, i, 0)),
            pl.BlockSpec((BLK, 1), lambda i: (i, 0)),
            pl.BlockSpec((5, H), lambda i: (0, 0)),
            pl.BlockSpec((H, H), lambda i: (0, 0)),
        ],
        out_specs=pl.BlockSpec((BLK, H), lambda i: (i, 0)),
        out_shape=jax.ShapeDtypeStruct((N, H), jnp.float32),
    )(m, p, dinv2d, prm, w)


def _tc_final_body(m_ref, p_ref, dinv_ref, prm_ref, batch_ref, wl_ref,
                   out_ref, acc_ref, cnt_ref):
    i = pl.program_id(0)

    @pl.when(i == 0)
    def _():
        acc_ref[...] = jnp.zeros_like(acc_ref)
        cnt_ref[...] = jnp.zeros_like(cnt_ref)

    h = _combine_h(m_ref[...], p_ref[...], dinv_ref[...], prm_ref[...])
    oh = (batch_ref[...] ==
          lax.broadcasted_iota(jnp.int32, (1, G), 1)).astype(jnp.float32)
    acc_ref[...] += lax.dot_general(
        oh, h, (((0,), (0,)), ((), ())), preferred_element_type=jnp.float32)
    cnt_ref[...] += lax.dot_general(
        oh, jnp.ones((BLK, H), jnp.float32), (((0,), (0,)), ((), ())),
        preferred_element_type=jnp.float32)

    @pl.when(i == pl.num_programs(0) - 1)
    def _():
        pooled = acc_ref[...] / jnp.maximum(cnt_ref[...], 1.0)
        out_ref[...] = lax.dot_general(
            wl_ref[...], pooled, (((1,), (1,)), ((), ())),
            preferred_element_type=jnp.float32)


def _tc_final(m, p, dinv2d, prm, batch2d, wl):
    return pl.pallas_call(
        _tc_final_body,
        grid=(GRID,),
        in_specs=[
            pl.BlockSpec((BLK, H), lambda i: (i, 0)),
            pl.BlockSpec((NC, BLK, H), lambda i: (0, i, 0)),
            pl.BlockSpec((BLK, 1), lambda i: (i, 0)),
            pl.BlockSpec((5, H), lambda i: (0, 0)),
            pl.BlockSpec((BLK, 1), lambda i: (i, 0)),
            pl.BlockSpec((1, H), lambda i: (0, 0)),
        ],
        out_specs=pl.BlockSpec((1, G), lambda i: (0, 0)),
        out_shape=jax.ShapeDtypeStruct((1, G), jnp.float32),
        scratch_shapes=[
            pltpu.VMEM((G, H), jnp.float32),
            pltpu.VMEM((G, H), jnp.float32),
        ],
    )(m, p, dinv2d, prm, batch2d, wl)


def kernel(x, edge_index, batch,
           W0, b0, g0, bt0, rm0, rv0,
           W1, b1, g1, bt1, rm1, rv1,
           W2, b2, g2, bt2, rm2, rv2,
           Wl, bl):
    src = edge_index[0]
    dst = edge_index[1]
    srcm = src.reshape(NW, EPT)[:, :M * CHUNK].reshape(NW, M, CHUNK)
    dstm = dst.reshape(NW, EPT)[:, :M * CHUNK].reshape(NW, M, CHUNK)
    packed = jnp.stack([srcm, dstm], axis=2).reshape(NW * M, 2, CHUNK)

    degp = _deg_partials(packed, dst).reshape(NC, N)
    deg = degp[0] + degp[1] + 1.0
    dinv2d = lax.rsqrt(deg)[:, None]

    prm0 = jnp.stack([b0, g0, bt0, rm0, rv0])
    prm1 = jnp.stack([b1, g1, bt1, rm1, rv1])
    prm2 = jnp.stack([b2, g2, bt2, rm2, rv2])

    m0 = _tc0(x, W0, dinv2d)
    p0 = _scatter_partials(m0, packed, src, dst)
    m1 = _tc_combine(m0, p0, dinv2d, prm0, W1)
    p1 = _scatter_partials(m1, packed, src, dst)
    m2 = _tc_combine(m1, p1, dinv2d, prm1, W2)
    p2 = _scatter_partials(m2, packed, src, dst)

    batch2d = batch[:, None]
    out = _tc_final(m2, p2, dinv2d, prm2, batch2d, Wl)
    return out.reshape(G, 1) + bl
